# baseline (speedup 1.0000x reference)
import functools

import jax
import jax.numpy as jnp
from jax import lax
from jax.experimental import pallas as pl
from jax.experimental.pallas import tpu as pltpu
from jax.experimental.pallas import tpu_sc as plsc

BATCH = 1024
SEQ = 200
HIDDEN = 128
L = 16
NC, NS = 2, 16
NW = NC * NS
NROWS = BATCH * SEQ
RPW = NROWS // NW
CH = 64
NCHUNK = RPW // CH
NBUF = 5
DEPTH = 3
NCOMBO = 2 * SEQ
JV = HIDDEN // L
UNROLL = 4
CHUNK_BYTES = CH * HIDDEN * 4


def _lane_splat(v, lane_idx):
    dn = lax.GatherDimensionNumbers(
        offset_dims=(), collapsed_slice_dims=(0,), start_index_map=(0,))
    return lax.gather(v, lane_idx[:, None], dn, (1,),
                      mode=lax.GatherScatterMode.PROMISE_IN_BOUNDS)


def _sc_body(tok3, segf, ttab, stf, ptf, gam, bet, out,
             combo, rows0, rows1, rows2, rows3, rows4, idxall,
             cball, gv, bv, stv, gsem, wsem, ssem):
    wid = lax.axis_index("s") * NC + lax.axis_index("c")
    base0 = wid * RPW
    rows = (rows0, rows1, rows2, rows3, rows4)

    pltpu.sync_copy(gam, gv)
    pltpu.sync_copy(bet, bv)
    pltpu.sync_copy(stf, stv)
    pltpu.sync_copy(tok3.at[wid], idxall)
    pltpu.sync_copy(ptf.at[pl.ds(0, SEQ * HIDDEN)], combo.at[pl.ds(0, SEQ * HIDDEN)])
    pltpu.sync_copy(ptf.at[pl.ds(0, SEQ * HIDDEN)],
                    combo.at[pl.ds(SEQ * HIDDEN, SEQ * HIDDEN)])

    def _addseg(p, carry):
        off = p * HIDDEN
        for s_ in range(2):
            for j in range(JV):
                sl = pl.ds(s_ * SEQ * HIDDEN + off + j * L, L)
                combo[sl] = combo[sl] + stv[pl.ds(s_ * HIDDEN + j * L, L)]
        return carry

    lax.fori_loop(0, SEQ, _addseg, 0)

    pltpu.async_copy(segf.at[pl.ds(base0, RPW)], cball, ssem).wait()

    def _cb(g, carry):
        i16 = lax.iota(jnp.int32, L) + (base0 + g * L)
        pos = lax.rem(i16, SEQ)
        sg = cball[pl.ds(g * L, L)]
        cball[pl.ds(g * L, L)] = (sg * SEQ + pos) * HIDDEN
        return carry

    lax.fori_loop(0, RPW // L, _cb, 0)

    col = [lax.iota(jnp.int32, L) + j * L for j in range(JV)]
    gvr = [gv[pl.ds(j * L, L)] for j in range(JV)]
    bvr = [bv[pl.ds(j * L, L)] for j in range(JV)]
    inv_h = jnp.float32(1.0 / HIDDEN)
    lane15 = jnp.full((L,), L - 1, jnp.int32)

    def _gather_start(k, b):
        pltpu.async_copy(ttab.at[idxall.at[k]], rows[b], gsem)

    def _gather_wait(b):
        pltpu.make_async_copy(ttab.at[pl.ds(0, CH)], rows[b], gsem).wait()

    def _write_start(k, b):
        pltpu.async_copy(rows[b], out.at[pl.ds(base0 + k * CH, CH)], wsem)

    def _write_wait(b):
        pltpu.make_async_copy(rows[b], out.at[pl.ds(0, CH)], wsem).wait()

    for m in range(DEPTH):
        _gather_start(m, m)

    def _row_block(b, k, i):
        for u in range(UNROLL):
            r = i + u
            cb = plsc.load_gather(
                cball, [jnp.broadcast_to(k * CH + r, (L,)).astype(jnp.int32)])
            xs = []
            for j in range(JV):
                te = rows[b][r, pl.ds(j * L, L)]
                cm = plsc.load_gather(combo, [cb + col[j]])
                xs.append(te + cm)
            s = ((xs[0] + xs[1]) + (xs[2] + xs[3])) + ((xs[4] + xs[5]) + (xs[6] + xs[7]))
            q0 = xs[0] * xs[0] + xs[1] * xs[1]
            q1 = xs[2] * xs[2] + xs[3] * xs[3]
            q2 = xs[4] * xs[4] + xs[5] * xs[5]
            q3 = xs[6] * xs[6] + xs[7] * xs[7]
            q = (q0 + q1) + (q2 + q3)
            mean = _lane_splat(plsc.cumsum(s), lane15) * inv_h
            msq = _lane_splat(plsc.cumsum(q), lane15) * inv_h
            a = msq - mean * mean + jnp.float32(1e-3)
            bits = plsc.bitcast(a, jnp.int32)
            y = plsc.bitcast(jnp.int32(0x5F3759DF) - (bits >> 1), jnp.float32)
            half_a = a * jnp.float32(0.5)
            for _ in range(2):
                y = y * (jnp.float32(1.5) - half_a * y * y)
            c0 = jnp.float32(0.0) - mean * y
            for j in range(JV):
                z = xs[j] * y + c0
                rows[b][r, pl.ds(j * L, L)] = z * gvr[j] + bvr[j]

    def _super(t, carry):
        for b in range(NBUF):
            k = NBUF * t + b
            _gather_wait(b)

            @pl.when(k + DEPTH < NCHUNK)
            def _():
                @pl.when(k >= NBUF - DEPTH)
                def _():
                    _write_wait((b + DEPTH) % NBUF)
                _gather_start(k + DEPTH, (b + DEPTH) % NBUF)

            def _rb(i, c):
                _row_block(b, k, i * UNROLL)
                return c

            lax.fori_loop(0, CH // UNROLL, _rb, 0)
            _write_start(k, b)
        return carry

    lax.fori_loop(0, NCHUNK // NBUF, _super, 0)
    for b in range(NBUF):
        _write_wait(b)


@jax.jit
def _run(tok3, segf, ttab, stf, ptf, gam, bet):
    mesh = plsc.VectorSubcoreMesh(core_axis_name="c", subcore_axis_name="s")
    f = pl.kernel(
        _sc_body,
        out_type=jax.ShapeDtypeStruct((NROWS, HIDDEN), jnp.float32),
        mesh=mesh,
        compiler_params=pltpu.CompilerParams(needs_layout_passes=False),
        scratch_types=[
            pltpu.VMEM((NCOMBO * HIDDEN,), jnp.float32),
            pltpu.VMEM((CH, HIDDEN), jnp.float32),
            pltpu.VMEM((CH, HIDDEN), jnp.float32),
            pltpu.VMEM((CH, HIDDEN), jnp.float32),
            pltpu.VMEM((CH, HIDDEN), jnp.float32),
            pltpu.VMEM((CH, HIDDEN), jnp.float32),
            pltpu.VMEM((NCHUNK, CH), jnp.int32),
            pltpu.VMEM((RPW,), jnp.int32),
            pltpu.VMEM((HIDDEN,), jnp.float32),
            pltpu.VMEM((HIDDEN,), jnp.float32),
            pltpu.VMEM((2 * HIDDEN,), jnp.float32),
            pltpu.SemaphoreType.DMA,
            pltpu.SemaphoreType.DMA,
            pltpu.SemaphoreType.DMA,
        ],
    )
    return f(tok3, segf, ttab, stf, ptf, gam, bet)


def kernel(token, segment, token_table, segment_table, position_table, gamma, beta):
    tok3 = token.reshape(NW, NCHUNK, CH).astype(jnp.int32)
    segf = segment.reshape(-1).astype(jnp.int32)
    stf = segment_table.reshape(-1)
    ptf = position_table.reshape(-1)
    out = _run(tok3, segf, token_table, stf, ptf, gamma, beta)
    return out.reshape(BATCH, SEQ, HIDDEN)


# speedup vs baseline: 1.9702x; 1.9702x over previous
import functools

import jax
import jax.numpy as jnp
from jax import lax
from jax.experimental import pallas as pl
from jax.experimental.pallas import tpu as pltpu
from jax.experimental.pallas import tpu_sc as plsc

BATCH = 1024
SEQ = 200
HIDDEN = 128
L = 16
NC, NS = 2, 16
NW = NC * NS
NROWS = BATCH * SEQ
RPW = NROWS // NW
CH = 64
NCHUNK = RPW // CH
NBUF = 5
DEPTH = 3
NCOMBO = 2 * SEQ
JV = HIDDEN // L
UNROLL = 4
CHUNK_BYTES = CH * HIDDEN * 4


def _lane_splat(v, lane_idx):
    dn = lax.GatherDimensionNumbers(
        offset_dims=(), collapsed_slice_dims=(0,), start_index_map=(0,))
    return lax.gather(v, lane_idx[:, None], dn, (1,),
                      mode=lax.GatherScatterMode.PROMISE_IN_BOUNDS)


def _sc_body(tok3, segf, ttab, stf, ptf, gam, bet, out,
             combo, rows0, rows1, rows2, rows3, rows4, idxall,
             cball, gv, bv, stv, gsem, wsem, ssem):
    wid = lax.axis_index("s") * NC + lax.axis_index("c")
    base0 = wid * RPW
    rows = (rows0, rows1, rows2, rows3, rows4)

    pltpu.sync_copy(gam, gv)
    pltpu.sync_copy(bet, bv)
    pltpu.sync_copy(stf, stv)
    pltpu.sync_copy(tok3.at[wid], idxall)
    pltpu.sync_copy(ptf.at[pl.ds(0, SEQ * HIDDEN)], combo.at[pl.ds(0, SEQ * HIDDEN)])
    pltpu.sync_copy(ptf.at[pl.ds(0, SEQ * HIDDEN)],
                    combo.at[pl.ds(SEQ * HIDDEN, SEQ * HIDDEN)])

    def _addseg(p, carry):
        off = p * HIDDEN
        for s_ in range(2):
            for j in range(JV):
                sl = pl.ds(s_ * SEQ * HIDDEN + off + j * L, L)
                combo[sl] = combo[sl] + stv[pl.ds(s_ * HIDDEN + j * L, L)]
        return carry

    lax.fori_loop(0, SEQ, _addseg, 0)

    pltpu.async_copy(segf.at[pl.ds(base0, RPW)], cball, ssem).wait()

    def _cb(g, carry):
        i16 = lax.iota(jnp.int32, L) + (base0 + g * L)
        pos = lax.rem(i16, SEQ)
        sg = cball[pl.ds(g * L, L)]
        cball[pl.ds(g * L, L)] = (sg * SEQ + pos) * HIDDEN
        return carry

    lax.fori_loop(0, RPW // L, _cb, 0)

    col = [lax.iota(jnp.int32, L) + j * L for j in range(JV)]
    gvr = [gv[pl.ds(j * L, L)] for j in range(JV)]
    bvr = [bv[pl.ds(j * L, L)] for j in range(JV)]
    inv_h = jnp.float32(1.0 / HIDDEN)
    lane15 = jnp.full((L,), L - 1, jnp.int32)

    def _gather_start(k, b):
        pltpu.async_copy(ttab.at[idxall.at[k]], rows[b], gsem)

    def _gather_wait(b):
        pltpu.make_async_copy(ttab.at[pl.ds(0, CH)], rows[b], gsem).wait()

    def _write_start(k, b):
        pltpu.async_copy(rows[b], out.at[pl.ds(base0 + k * CH, CH)], wsem)

    def _write_wait(b):
        pltpu.make_async_copy(rows[b], out.at[pl.ds(0, CH)], wsem).wait()

    for m in range(DEPTH):
        _gather_start(m, m)

    def _row_block(b, k, i):
        stats = []
        xs_all = []
        for u in range(UNROLL):
            r = i + u
            cb = plsc.load_gather(
                cball, [jnp.broadcast_to(k * CH + r, (L,)).astype(jnp.int32)])
            xs = []
            for j in range(JV):
                te = rows[b][r, pl.ds(j * L, L)]
                cm = plsc.load_gather(combo, [cb + col[j]])
                xs.append(te + cm)
            s = ((xs[0] + xs[1]) + (xs[2] + xs[3])) + ((xs[4] + xs[5]) + (xs[6] + xs[7]))
            q0 = xs[0] * xs[0] + xs[1] * xs[1]
            q1 = xs[2] * xs[2] + xs[3] * xs[3]
            q2 = xs[4] * xs[4] + xs[5] * xs[5]
            q3 = xs[6] * xs[6] + xs[7] * xs[7]
            q = (q0 + q1) + (q2 + q3)
            xs_all.append(xs)
            stats.append((s, q))
        for u in range(UNROLL):
            r = i + u
            s, q = stats[u]
            xs = xs_all[u]
            mean = _lane_splat(plsc.cumsum(s), lane15) * inv_h
            msq = _lane_splat(plsc.cumsum(q), lane15) * inv_h
            a = msq - mean * mean + jnp.float32(1e-3)
            bits = plsc.bitcast(a, jnp.int32)
            y = plsc.bitcast(jnp.int32(0x5F3759DF) - (bits >> 1), jnp.float32)
            half_a = a * jnp.float32(0.5)
            for _ in range(2):
                y = y * (jnp.float32(1.5) - half_a * y * y)
            c0 = jnp.float32(0.0) - mean * y
            for j in range(JV):
                z = xs[j] * y + c0
                rows[b][r, pl.ds(j * L, L)] = z * gvr[j] + bvr[j]

    def _super(t, carry):
        for b in range(NBUF):
            k = NBUF * t + b
            _gather_wait(b)

            @pl.when(k + DEPTH < NCHUNK)
            def _():
                @pl.when(k >= NBUF - DEPTH)
                def _():
                    _write_wait((b + DEPTH) % NBUF)
                _gather_start(k + DEPTH, (b + DEPTH) % NBUF)

            def _rb(i, c):
                _row_block(b, k, i * UNROLL)
                return c

            lax.fori_loop(0, CH // UNROLL, _rb, 0)
            _write_start(k, b)
        return carry

    lax.fori_loop(0, NCHUNK // NBUF, _super, 0)
    for b in range(NBUF):
        _write_wait(b)


@jax.jit
def _run(tok3, segf, ttab, stf, ptf, gam, bet):
    mesh = plsc.VectorSubcoreMesh(core_axis_name="c", subcore_axis_name="s")
    f = pl.kernel(
        _sc_body,
        out_type=jax.ShapeDtypeStruct((NROWS, HIDDEN), jnp.float32),
        mesh=mesh,
        compiler_params=pltpu.CompilerParams(needs_layout_passes=False),
        scratch_types=[
            pltpu.VMEM((NCOMBO * HIDDEN,), jnp.float32),
            pltpu.VMEM((CH, HIDDEN), jnp.float32),
            pltpu.VMEM((CH, HIDDEN), jnp.float32),
            pltpu.VMEM((CH, HIDDEN), jnp.float32),
            pltpu.VMEM((CH, HIDDEN), jnp.float32),
            pltpu.VMEM((CH, HIDDEN), jnp.float32),
            pltpu.VMEM((NCHUNK, CH), jnp.int32),
            pltpu.VMEM((RPW,), jnp.int32),
            pltpu.VMEM((HIDDEN,), jnp.float32),
            pltpu.VMEM((HIDDEN,), jnp.float32),
            pltpu.VMEM((2 * HIDDEN,), jnp.float32),
            pltpu.SemaphoreType.DMA,
            pltpu.SemaphoreType.DMA,
            pltpu.SemaphoreType.DMA,
        ],
    )
    return f(tok3, segf, ttab, stf, ptf, gam, bet)


def kernel(token, segment, token_table, segment_table, position_table, gamma, beta):
    tok3 = token.reshape(NW, NCHUNK, CH).astype(jnp.int32)
    segf = segment.reshape(-1).astype(jnp.int32)
    stf = segment_table.reshape(-1)
    ptf = position_table.reshape(-1)
    out = _run(tok3, segf, token_table, stf, ptf, gamma, beta)
    return out.reshape(BATCH, SEQ, HIDDEN)
